# baseline (speedup 1.0000x reference)
import functools

import jax
import jax.numpy as jnp
from jax import lax
from jax.experimental import pallas as pl
from jax.experimental.pallas import tpu as pltpu
from jax.experimental.pallas import tpu_sc as plsc

V = 100000
D = 64
L = 50
B = 1024
EPS = 1e-5

NC = 2
NS = 16
NW = NC * NS
SEQ_PER_W = B // NW
IDX_PER_W = SEQ_PER_W * L
CHUNK = 80
NCHUNK = IDX_PER_W // CHUNK

VT = 2048


def _sc_pool(idx3, table):
    mesh = plsc.VectorSubcoreMesh(core_axis_name="c", subcore_axis_name="s")

    @functools.partial(
        pl.kernel,
        mesh=mesh,
        out_type=jax.ShapeDtypeStruct((B, D), jnp.float32),
        compiler_params=pltpu.CompilerParams(use_tc_tiling_on_sc=False),
        scratch_types=[
            pltpu.VMEM((NCHUNK, CHUNK), jnp.int32),
            pltpu.VMEM((IDX_PER_W, D), jnp.float32),
            pltpu.VMEM((SEQ_PER_W, D), jnp.float32),
            pltpu.SemaphoreType.DMA,
        ],
    )
    def pool(idx_hbm, table_hbm, out_hbm, idx_v, rows_v, out_v, sem):
        wid = lax.axis_index("s") * NC + lax.axis_index("c")
        pltpu.sync_copy(idx_hbm.at[wid], idx_v)
        copies = [
            pltpu.async_copy(
                table_hbm.at[idx_v.at[j]],
                rows_v.at[pl.ds(j * CHUNK, CHUNK)],
                sem,
            )
            for j in range(NCHUNK)
        ]
        for cp in copies:
            cp.wait()

        def seq_body(sq, carry):
            base = sq * L
            accs = tuple(jnp.zeros((16,), jnp.float32) for _ in range(D // 16))
            def l_body(l, a):
                r = base + l
                return tuple(a[d] + rows_v[r, pl.ds(d * 16, 16)]
                             for d in range(D // 16))
            accs = lax.fori_loop(0, L, l_body, accs)
            for d in range(D // 16):
                out_v[sq, pl.ds(d * 16, 16)] = accs[d] * (1.0 / L)
            return carry

        lax.fori_loop(0, SEQ_PER_W, seq_body, 0)
        pltpu.sync_copy(out_v, out_hbm.at[pl.ds(wid * SEQ_PER_W, SEQ_PER_W)])

    return pool(idx3, table)


def _tc_decode(pooled, pos_p, gb, W, b):
    grid = pl.cdiv(V, VT)

    def body(pooled_ref, pos_ref, gb_ref, w_ref, b_ref, out_ref, normed_ref):
        @pl.when(pl.program_id(0) == 0)
        def _():
            pos_mean = jnp.sum(pos_ref[...], axis=0) * (1.0 / L)
            p = pooled_ref[...] + pos_mean[None, :]
            mu = jnp.mean(p, axis=1, keepdims=True)
            var = jnp.mean(jnp.square(p - mu), axis=1, keepdims=True)
            inv = lax.rsqrt(var + EPS)
            normed_ref[...] = ((p - mu) * inv * gb_ref[0][None, :]
                               + gb_ref[1][None, :])

        out_ref[...] = (
            jnp.dot(normed_ref[...], w_ref[...],
                    preferred_element_type=jnp.float32)
            + b_ref[...][None, :]
        )

    return pl.pallas_call(
        body,
        grid=(grid,),
        in_specs=[
            pl.BlockSpec((B, D), lambda j: (0, 0)),
            pl.BlockSpec((56, D), lambda j: (0, 0)),
            pl.BlockSpec((8, D), lambda j: (0, 0)),
            pl.BlockSpec((D, VT), lambda j: (0, j)),
            pl.BlockSpec((VT,), lambda j: (j,)),
        ],
        out_specs=pl.BlockSpec((B, VT), lambda j: (0, j)),
        out_shape=jax.ShapeDtypeStruct((B, V), jnp.float32),
        scratch_shapes=[pltpu.VMEM((B, D), jnp.float32)],
        compiler_params=pltpu.CompilerParams(
            dimension_semantics=("arbitrary",),
        ),
    )(pooled, pos_p, gb, W, b)


def kernel(x, token_emb, pos_emb, gamma, beta, W, b):
    idx3 = x.astype(jnp.int32).reshape(NW, NCHUNK, CHUNK)
    pooled = _sc_pool(idx3, token_emb)
    pos_p = jnp.pad(pos_emb, ((0, 56 - L), (0, 0)))
    gb = jnp.concatenate(
        [gamma[None, :], beta[None, :], jnp.zeros((6, D), jnp.float32)], axis=0)
    return _tc_decode(pooled, pos_p, gb, W, b)
